# Initial kernel scaffold; baseline (speedup 1.0000x reference)
import jax
import jax.numpy as jnp
from jax.experimental import pallas as pl


def kernel(x1, edge_index1, batch1, x2, edge_index2, batch2, emb_W, emb_b, layers, Wg, bg, Wv, bv):
    raise NotImplementedError("write your pallas kernel here")


# speedup vs baseline: 1.8086x; 1.8086x over previous
import functools

import jax
import jax.numpy as jnp
from jax import lax
from jax.experimental import pallas as pl
from jax.experimental.pallas import tpu as pltpu
from jax.experimental.pallas import tpu_sc as plsc

F32 = jnp.float32
BN = 256
G = 64
NEG = -1e9
ECH = 128
NWORK = 32
FP = 2


def _build_sc_messages(NP, E_pad, H):
    EW = E_pad // NWORK
    NCH = EW // ECH
    SLAB = NP // 16
    FW = 2 * H // FP
    NF = FW // 16
    mesh = plsc.VectorSubcoreMesh(core_axis_name="c", subcore_axis_name="s")
    out_type = [jax.ShapeDtypeStruct((2, FP, NP, FW), F32),
                jax.ShapeDtypeStruct((2, FP, NP, FW), F32)]
    scratch = [
        pltpu.VMEM((ECH,), jnp.int32),
        pltpu.VMEM((ECH,), jnp.int32),
        pltpu.VMEM((ECH, FW), F32),
        pltpu.VMEM((ECH, FW), F32),
        pltpu.VMEM((64, FW), F32),
        pltpu.VMEM_SHARED((NP, FW), F32),
        pltpu.SemaphoreType.DMA,
        pltpu.SemaphoreType.DMA,
    ]

    @functools.partial(pl.kernel, mesh=mesh, out_type=out_type,
                       scratch_types=scratch)
    def sc_msg(A1, B1, A2, B2, s1, d1, s2, d2, out1, out2,
               sidx, didx, rowsA, rowsB, zbuf, S_sh, semA, semB):
        cid = lax.axis_index("c")
        sid = lax.axis_index("s")
        wid = sid * 2 + cid
        base = wid * EW
        z16 = jnp.zeros((16,), F32)

        def _zb(r, carry):
            for f in range(NF):
                zbuf[r, pl.ds(16 * f, 16)] = z16
            return carry
        lax.fori_loop(0, 64, _zb, 0)

        def do_side(A, B, srcs, dsts, out):
            for p in range(FP):
                r0 = sid * SLAB

                def _zero(k, carry):
                    pltpu.sync_copy(zbuf, S_sh.at[pl.ds(r0 + k * 64, 64)])
                    return carry
                lax.fori_loop(0, SLAB // 64, _zero, 0)
                plsc.subcore_barrier()

                def _chunk(c, carry):
                    off = base + c * ECH
                    pltpu.sync_copy(srcs.at[pl.ds(off, ECH)], sidx)
                    pltpu.sync_copy(dsts.at[pl.ds(off, ECH)], didx)
                    cpA = pltpu.async_copy(A.at[p].at[sidx], rowsA, semA)
                    cpB = pltpu.async_copy(B.at[p].at[didx], rowsB, semB)
                    cpA.wait()
                    cpB.wait()

                    def _edge(e, ecarry):
                        for f in range(NF):
                            sl = pl.ds(16 * f, 16)
                            rowsA[e, sl] = jnp.maximum(
                                rowsA[e, sl] + rowsB[e, sl], 0.0)
                        return ecarry
                    lax.fori_loop(0, ECH, _edge, 0)
                    pltpu.sync_copy(rowsA, S_sh.at[didx], add=True)
                    return carry
                lax.fori_loop(0, NCH, _chunk, 0)
                plsc.subcore_barrier()

                def _flush(k, carry):
                    rr = r0 + k * 64
                    pltpu.sync_copy(S_sh.at[pl.ds(rr, 64)],
                                    out.at[cid, p, pl.ds(rr, 64)])
                    return carry
                lax.fori_loop(0, SLAB // 64, _flush, 0)
                plsc.subcore_barrier()

        do_side(A1, B1, s1, d1, out1)
        do_side(A2, B2, s2, d2, out2)

    return sc_msg


def _build_sc_deg(NP, E_pad):
    EW = E_pad // NWORK
    NCH = EW // ECH
    SLAB = NP // 16
    mesh = plsc.VectorSubcoreMesh(core_axis_name="c", subcore_axis_name="s")
    out_type = [jax.ShapeDtypeStruct((2, NP, 16), F32),
                jax.ShapeDtypeStruct((2, NP, 16), F32)]
    scratch = [
        pltpu.VMEM((ECH,), jnp.int32),
        pltpu.VMEM((ECH, 16), F32),
        pltpu.VMEM((64, 16), F32),
        pltpu.VMEM_SHARED((NP, 16), F32),
    ]

    @functools.partial(pl.kernel, mesh=mesh, out_type=out_type,
                       scratch_types=scratch)
    def sc_deg(d1, d2, cnt1, cnt2, didx, obuf, zbufc, C_sh):
        cid = lax.axis_index("c")
        sid = lax.axis_index("s")
        wid = sid * 2 + cid
        base = wid * EW
        z16 = jnp.zeros((16,), F32)
        o16 = jnp.ones((16,), F32)

        def _init(r, carry):
            obuf[r, pl.ds(0, 16)] = o16
            return carry
        lax.fori_loop(0, ECH, _init, 0)

        def _zb(r, carry):
            zbufc[r, pl.ds(0, 16)] = z16
            return carry
        lax.fori_loop(0, 64, _zb, 0)

        for dsts, cnt_out in ((d1, cnt1), (d2, cnt2)):
            r0 = sid * SLAB

            def _zero(k, carry):
                pltpu.sync_copy(zbufc, C_sh.at[pl.ds(r0 + k * 64, 64)])
                return carry
            lax.fori_loop(0, SLAB // 64, _zero, 0)
            plsc.subcore_barrier()

            def _chunk(c, carry):
                off = base + c * ECH
                pltpu.sync_copy(dsts.at[pl.ds(off, ECH)], didx)
                pltpu.sync_copy(obuf, C_sh.at[didx], add=True)
                return carry
            lax.fori_loop(0, NCH, _chunk, 0)
            plsc.subcore_barrier()

            def _flush(k, carry):
                rr = r0 + k * 64
                pltpu.sync_copy(C_sh.at[pl.ds(rr, 64)],
                                cnt_out.at[cid, pl.ds(rr, 64)])
                return carry
            lax.fori_loop(0, SLAB // 64, _flush, 0)
            plsc.subcore_barrier()

    return sc_deg


def _embed_call(xp, emb_W, emb_b2, Wm1, bm1_2, N, NP, H, interpret=False):
    NBn = NP // BN

    def body(x_ref, W_ref, b_ref, Wm1_ref, bm1_ref, h_ref, A_ref, B_ref):
        i = pl.program_id(0)
        h = jnp.dot(x_ref[...], W_ref[...], preferred_element_type=F32)
        h = h + b_ref[...]
        rows = i * BN + lax.broadcasted_iota(jnp.int32, (BN, 1), 0)
        h = jnp.where(rows < N, h, 0.0)
        h_ref[...] = h
        Ap = jnp.dot(h, Wm1_ref[0:H, :], preferred_element_type=F32)
        Bp = jnp.dot(h, Wm1_ref[H:2 * H, :], preferred_element_type=F32)
        Bp = Bp + bm1_ref[...]
        FW = 2 * H // FP
        for k in range(FP):
            A_ref[k] = Ap[:, k * FW:(k + 1) * FW]
            B_ref[k] = Bp[:, k * FW:(k + 1) * FW]

    FW = 2 * H // FP
    return pl.pallas_call(
        body,
        grid=(NBn,),
        in_specs=[
            pl.BlockSpec((BN, H), lambda i: (i, 0)),
            pl.BlockSpec((H, H), lambda i: (0, 0)),
            pl.BlockSpec((1, H), lambda i: (0, 0)),
            pl.BlockSpec((2 * H, 2 * H), lambda i: (0, 0)),
            pl.BlockSpec((1, 2 * H), lambda i: (0, 0)),
        ],
        out_specs=[
            pl.BlockSpec((BN, H), lambda i: (i, 0)),
            pl.BlockSpec((FP, BN, FW), lambda i: (0, i, 0)),
            pl.BlockSpec((FP, BN, FW), lambda i: (0, i, 0)),
        ],
        out_shape=[
            jax.ShapeDtypeStruct((NP, H), F32),
            jax.ShapeDtypeStruct((FP, NP, FW), F32),
            jax.ShapeDtypeStruct((FP, NP, FW), F32),
        ],
        interpret=interpret,
    )(xp, emb_W, emb_b2, Wm1, bm1_2)


def _counts_call(b1r, b2r, h1p, h2p, NP, H, interpret=False):
    NBn = NP // BN

    def body(b1_ref, b2_ref, h1_ref, h2_ref, c1_ref, c2_ref, s1_ref, s2_ref):
        i = pl.program_id(0)

        @pl.when(i == 0)
        def _():
            c1_ref[...] = jnp.zeros_like(c1_ref)
            c2_ref[...] = jnp.zeros_like(c2_ref)
            s1_ref[...] = jnp.zeros_like(s1_ref)
            s2_ref[...] = jnp.zeros_like(s2_ref)

        gids = lax.broadcasted_iota(jnp.int32, (BN, G), 1)
        for b_ref, h_ref, c_ref, s_ref in ((b1_ref, h1_ref, c1_ref, s1_ref),
                                           (b2_ref, h2_ref, c2_ref, s2_ref)):
            oh = (b_ref[0, 0, :][:, None] == gids).astype(F32)
            c_ref[0:1, :] = c_ref[0:1, :] + jnp.sum(oh, axis=0)[None, :]
            s_ref[0:1, :] = s_ref[0:1, :] + jnp.sum(h_ref[...], axis=0)[None, :]

    return pl.pallas_call(
        body,
        grid=(NBn,),
        in_specs=[
            pl.BlockSpec((1, 1, BN), lambda i: (i, 0, 0)),
            pl.BlockSpec((1, 1, BN), lambda i: (i, 0, 0)),
            pl.BlockSpec((BN, H), lambda i: (i, 0)),
            pl.BlockSpec((BN, H), lambda i: (i, 0)),
        ],
        out_specs=[
            pl.BlockSpec((8, G), lambda i: (0, 0)),
            pl.BlockSpec((8, G), lambda i: (0, 0)),
            pl.BlockSpec((8, H), lambda i: (0, 0)),
            pl.BlockSpec((8, H), lambda i: (0, 0)),
        ],
        out_shape=[
            jax.ShapeDtypeStruct((8, G), F32),
            jax.ShapeDtypeStruct((8, G), F32),
            jax.ShapeDtypeStruct((8, H), F32),
            jax.ShapeDtypeStruct((8, H), F32),
        ],
        interpret=interpret,
    )(b1r, b2r, h1p, h2p)


def _attn_specs(NP, H, NBn):
    def h1_idx(i, j, jlo, jhi):
        return (i, 0)

    def _jc(i, j, jlo, jhi):
        lo = jlo[i]
        hi = jnp.maximum(jhi[i], lo)
        return jnp.clip(j, lo, hi)

    def h2_idx(i, j, jlo, jhi):
        return (_jc(i, j, jlo, jhi), 0)

    def b1_idx(i, j, jlo, jhi):
        return (i, 0, 0)

    def b2_idx(i, j, jlo, jhi):
        return (_jc(i, j, jlo, jhi), 0, 0)

    return h1_idx, h2_idx, b1_idx, b2_idx, _jc


def _passA_call(jlo, jhi, h1p, h2p, b1r, b2r, NP, H, interpret=False):
    NBn = NP // BN
    h1_idx, h2_idx, b1_idx, b2_idx, _ = _attn_specs(NP, H, NBn)

    def body(jlo_ref, jhi_ref, h1_ref, h2_ref, b1_ref, b2_ref, M1_ref, M2_ref):
        i = pl.program_id(0)
        j = pl.program_id(1)

        @pl.when((i == 0) & (j == 0))
        def _():
            M1_ref[...] = jnp.full_like(M1_ref, NEG)
            M2_ref[...] = jnp.full_like(M2_ref, NEG)

        lo = jlo_ref[i]
        hi = jhi_ref[i]

        @pl.when((j >= lo) & (j <= hi))
        def _():
            L = lax.dot_general(h1_ref[...], h2_ref[...],
                                (((1,), (1,)), ((), ())),
                                preferred_element_type=F32)
            mask = b1_ref[0, 0, :][:, None] == b2_ref[0, 0, :][None, :]
            Lm = jnp.where(mask, L, NEG)
            rmax = jnp.max(Lm, axis=1)
            cmax = jnp.max(Lm, axis=0)
            M1_ref[pl.ds(i, 1), :] = jnp.maximum(M1_ref[pl.ds(i, 1), :],
                                                 rmax[None, :])
            M2_ref[pl.ds(j, 1), :] = jnp.maximum(M2_ref[pl.ds(j, 1), :],
                                                 cmax[None, :])

    grid_spec = pltpu.PrefetchScalarGridSpec(
        num_scalar_prefetch=2,
        grid=(NBn, NBn),
        in_specs=[
            pl.BlockSpec((BN, H), h1_idx),
            pl.BlockSpec((BN, H), h2_idx),
            pl.BlockSpec((1, 1, BN), b1_idx),
            pl.BlockSpec((1, 1, BN), b2_idx),
        ],
        out_specs=[
            pl.BlockSpec((NBn, BN), lambda i, j, jlo, jhi: (0, 0)),
            pl.BlockSpec((NBn, BN), lambda i, j, jlo, jhi: (0, 0)),
        ],
    )
    return pl.pallas_call(
        body,
        grid_spec=grid_spec,
        out_shape=[
            jax.ShapeDtypeStruct((NBn, BN), F32),
            jax.ShapeDtypeStruct((NBn, BN), F32),
        ],
        compiler_params=pltpu.CompilerParams(
            dimension_semantics=("arbitrary", "arbitrary")),
        interpret=interpret,
    )(jlo, jhi, h1p, h2p, b1r, b2r)


def _passB_call(jlo, jhi, h1p, h2p, b1r, b2r, M1r, M2r, NP, H, interpret=False):
    NBn = NP // BN
    h1_idx, h2_idx, b1_idx, b2_idx, _jc = _attn_specs(NP, H, NBn)

    def M2_idx(i, j, jlo, jhi):
        return (_jc(i, j, jlo, jhi), 0, 0)

    def body(jlo_ref, jhi_ref, h1_ref, h2_ref, b1_ref, b2_ref,
             M1_ref, M2_ref, O1_ref, O2_ref, S1_ref, S2_ref):
        i = pl.program_id(0)
        j = pl.program_id(1)

        @pl.when((i == 0) & (j == 0))
        def _():
            O1_ref[...] = jnp.zeros_like(O1_ref)
            O2_ref[...] = jnp.zeros_like(O2_ref)
            S1_ref[...] = jnp.zeros_like(S1_ref)
            S2_ref[...] = jnp.zeros_like(S2_ref)

        lo = jlo_ref[i]
        hi = jhi_ref[i]

        @pl.when((j >= lo) & (j <= hi))
        def _():
            h1b = h1_ref[...]
            h2b = h2_ref[...]
            L = lax.dot_general(h1b, h2b, (((1,), (1,)), ((), ())),
                                preferred_element_type=F32)
            mask = b1_ref[0, 0, :][:, None] == b2_ref[0, 0, :][None, :]
            m1 = M1_ref[0, 0, :]
            m2 = M2_ref[0, 0, :]
            P1 = jnp.exp(jnp.where(mask, L - m1[:, None], -1e30))
            P2 = jnp.exp(jnp.where(mask, L - m2[None, :], -1e30))
            S1_ref[pl.ds(i, 1), :] = (S1_ref[pl.ds(i, 1), :]
                                      + jnp.sum(P1, axis=1)[None, :])
            S2_ref[pl.ds(j, 1), :] = (S2_ref[pl.ds(j, 1), :]
                                      + jnp.sum(P2, axis=0)[None, :])
            O1_ref[pl.ds(i * BN, BN), :] = (
                O1_ref[pl.ds(i * BN, BN), :]
                + jnp.dot(P1, h2b, preferred_element_type=F32))
            O2_ref[pl.ds(j * BN, BN), :] = (
                O2_ref[pl.ds(j * BN, BN), :]
                + lax.dot_general(P2, h1b, (((0,), (0,)), ((), ())),
                                  preferred_element_type=F32))

    grid_spec = pltpu.PrefetchScalarGridSpec(
        num_scalar_prefetch=2,
        grid=(NBn, NBn),
        in_specs=[
            pl.BlockSpec((BN, H), h1_idx),
            pl.BlockSpec((BN, H), h2_idx),
            pl.BlockSpec((1, 1, BN), b1_idx),
            pl.BlockSpec((1, 1, BN), b2_idx),
            pl.BlockSpec((1, 1, BN), b1_idx),
            pl.BlockSpec((1, 1, BN), M2_idx),
        ],
        out_specs=[
            pl.BlockSpec((NP, H), lambda i, j, jlo, jhi: (0, 0)),
            pl.BlockSpec((NP, H), lambda i, j, jlo, jhi: (0, 0)),
            pl.BlockSpec((NBn, BN), lambda i, j, jlo, jhi: (0, 0)),
            pl.BlockSpec((NBn, BN), lambda i, j, jlo, jhi: (0, 0)),
        ],
    )
    return pl.pallas_call(
        body,
        grid_spec=grid_spec,
        out_shape=[
            jax.ShapeDtypeStruct((NP, H), F32),
            jax.ShapeDtypeStruct((NP, H), F32),
            jax.ShapeDtypeStruct((NBn, BN), F32),
            jax.ShapeDtypeStruct((NBn, BN), F32),
        ],
        compiler_params=pltpu.CompilerParams(
            dimension_semantics=("arbitrary", "arbitrary")),
        interpret=interpret,
    )(jlo, jhi, h1p, h2p, b1r, b2r, M1r, M2r)


def _update_call(hp, Sp, O1, S1r, degr, br, cntO, csO, Wm2, bm2_2,
                 Wu1, bu1_2, Wu2, bu2_2, Wm1n, bm1n_2, N, NP, H,
                 interpret=False):
    NBn = NP // BN
    make_ab = Wm1n is not None

    def body(*refs):
        if make_ab:
            (h_ref, S_ref, O1_ref, S1_ref, deg_ref, b_ref, cnt_ref, cs_ref,
             Wm2_ref, bm2_ref, Wu1_ref, bu1_ref, Wu2_ref, bu2_ref,
             Wm1n_ref, bm1n_ref, hn_ref, A_ref, B_ref) = refs
        else:
            (h_ref, S_ref, O1_ref, S1_ref, deg_ref, b_ref, cnt_ref, cs_ref,
             Wm2_ref, bm2_ref, Wu1_ref, bu1_ref, Wu2_ref, bu2_ref,
             hn_ref) = refs
        i = pl.program_id(0)
        h = h_ref[...]
        sp = S_ref[...]
        m_pre = jnp.concatenate([sp[0, k] + sp[1, k] for k in range(FP)],
                                axis=1)
        deg = deg_ref[0, 0, :][:, None]
        m = (jnp.dot(m_pre, Wm2_ref[...], preferred_element_type=F32)
             + deg * bm2_ref[...])
        b = b_ref[0, 0, :]
        gids = lax.broadcasted_iota(jnp.int32, (BN, G), 1)
        oh = (b[:, None] == gids).astype(F32)
        cnt_row = jnp.dot(oh, cnt_ref[0:1, :].T,
                          preferred_element_type=F32)
        empty = cnt_row <= 0.5
        S1v = S1_ref[0, 0, :][:, None]
        att = O1_ref[...] / jnp.where(S1v > 0, S1v, 1.0)
        mean_other = cs_ref[0:1, :] / N
        mu = h - jnp.where(empty, mean_other, att)
        cat = jnp.concatenate([h, m, mu], axis=1)
        u = jnp.maximum(jnp.dot(cat, Wu1_ref[...],
                                preferred_element_type=F32) + bu1_ref[...],
                        0.0)
        u = jnp.dot(u, Wu2_ref[...], preferred_element_type=F32) + bu2_ref[...]
        hn = h + u
        rows = i * BN + lax.broadcasted_iota(jnp.int32, (BN, 1), 0)
        hn = jnp.where(rows < N, hn, 0.0)
        hn_ref[...] = hn
        if make_ab:
            Ap = jnp.dot(hn, Wm1n_ref[0:H, :], preferred_element_type=F32)
            Bp = jnp.dot(hn, Wm1n_ref[H:2 * H, :],
                         preferred_element_type=F32) + bm1n_ref[...]
            FW = 2 * H // FP
            for k in range(FP):
                A_ref[k] = Ap[:, k * FW:(k + 1) * FW]
                B_ref[k] = Bp[:, k * FW:(k + 1) * FW]

    FW = 2 * H // FP
    in_specs = [
        pl.BlockSpec((BN, H), lambda i: (i, 0)),
        pl.BlockSpec((2, FP, BN, FW), lambda i: (0, 0, i, 0)),
        pl.BlockSpec((BN, H), lambda i: (i, 0)),
        pl.BlockSpec((1, 1, BN), lambda i: (i, 0, 0)),
        pl.BlockSpec((1, 1, BN), lambda i: (i, 0, 0)),
        pl.BlockSpec((1, 1, BN), lambda i: (i, 0, 0)),
        pl.BlockSpec((8, G), lambda i: (0, 0)),
        pl.BlockSpec((8, H), lambda i: (0, 0)),
        pl.BlockSpec((2 * H, 2 * H), lambda i: (0, 0)),
        pl.BlockSpec((1, 2 * H), lambda i: (0, 0)),
        pl.BlockSpec((4 * H, 2 * H), lambda i: (0, 0)),
        pl.BlockSpec((1, 2 * H), lambda i: (0, 0)),
        pl.BlockSpec((2 * H, H), lambda i: (0, 0)),
        pl.BlockSpec((1, H), lambda i: (0, 0)),
    ]
    out_specs = [pl.BlockSpec((BN, H), lambda i: (i, 0))]
    out_shape = [jax.ShapeDtypeStruct((NP, H), F32)]
    args = [hp, Sp, O1, S1r, degr, br, cntO, csO, Wm2, bm2_2,
            Wu1, bu1_2, Wu2, bu2_2]
    if make_ab:
        in_specs += [pl.BlockSpec((2 * H, 2 * H), lambda i: (0, 0)),
                     pl.BlockSpec((1, 2 * H), lambda i: (0, 0))]
        out_specs += [pl.BlockSpec((FP, BN, FW), lambda i: (0, i, 0)),
                      pl.BlockSpec((FP, BN, FW), lambda i: (0, i, 0))]
        out_shape += [jax.ShapeDtypeStruct((FP, NP, FW), F32),
                      jax.ShapeDtypeStruct((FP, NP, FW), F32)]
        args += [Wm1n, bm1n_2]

    return pl.pallas_call(
        body,
        grid=(NBn,),
        in_specs=in_specs,
        out_specs=out_specs,
        out_shape=out_shape,
        interpret=interpret,
    )(*args)


def _readout_call(h1p, h2p, b1r, b2r, Wg, bg2, Wv, bv2, NP, H,
                  interpret=False):
    NBn = NP // BN

    def body(h1_ref, h2_ref, b1_ref, b2_ref, Wg_ref, bg_ref, Wv_ref, bv_ref,
             R1_ref, R2_ref):
        i = pl.program_id(0)

        @pl.when(i == 0)
        def _():
            R1_ref[...] = jnp.zeros_like(R1_ref)
            R2_ref[...] = jnp.zeros_like(R2_ref)

        gids = lax.broadcasted_iota(jnp.int32, (G, BN), 0)
        for h_ref, b_ref, R_ref in ((h1_ref, b1_ref, R1_ref),
                                    (h2_ref, b2_ref, R2_ref)):
            hb = h_ref[...]
            gate = jax.nn.sigmoid(
                jnp.dot(hb, Wg_ref[...], preferred_element_type=F32)
                + bg_ref[...])
            val = (jnp.dot(hb, Wv_ref[...], preferred_element_type=F32)
                   + bv_ref[...])
            gv = gate * val
            oh = (gids == b_ref[0, 0, :][None, :]).astype(F32)
            R_ref[...] = R_ref[...] + jnp.dot(oh, gv,
                                              preferred_element_type=F32)

    return pl.pallas_call(
        body,
        grid=(NBn,),
        in_specs=[
            pl.BlockSpec((BN, H), lambda i: (i, 0)),
            pl.BlockSpec((BN, H), lambda i: (i, 0)),
            pl.BlockSpec((1, 1, BN), lambda i: (i, 0, 0)),
            pl.BlockSpec((1, 1, BN), lambda i: (i, 0, 0)),
            pl.BlockSpec((H, H), lambda i: (0, 0)),
            pl.BlockSpec((1, H), lambda i: (0, 0)),
            pl.BlockSpec((H, H), lambda i: (0, 0)),
            pl.BlockSpec((1, H), lambda i: (0, 0)),
        ],
        out_specs=[
            pl.BlockSpec((G, H), lambda i: (0, 0)),
            pl.BlockSpec((G, H), lambda i: (0, 0)),
        ],
        out_shape=[
            jax.ShapeDtypeStruct((G, H), F32),
            jax.ShapeDtypeStruct((G, H), F32),
        ],
        interpret=interpret,
    )(h1p, h2p, b1r, b2r, Wg, bg2, Wv, bv2)


def kernel(x1, edge_index1, batch1, x2, edge_index2, batch2,
           emb_W, emb_b, layers, Wg, bg, Wv, bv):
    N, H = x1.shape
    E = edge_index1.shape[1]
    NP = -(-N // BN) * BN
    NBn = NP // BN
    padN = NP - N

    xp1 = jnp.pad(x1, ((0, padN), (0, 0)))
    xp2 = jnp.pad(x2, ((0, padN), (0, 0)))
    b1p = jnp.pad(batch1, (0, padN), constant_values=G + 63)
    b2p = jnp.pad(batch2, (0, padN), constant_values=G + 62)
    b1r = b1p.reshape(NBn, 1, BN)
    b2r = b2p.reshape(NBn, 1, BN)

    e1lo = b1p[::BN]
    e1hi = b1p[BN - 1::BN]
    e2lo = b2p[::BN]
    e2hi = b2p[BN - 1::BN]
    overlap = (e2hi[None, :] >= e1lo[:, None]) & (e2lo[None, :] <= e1hi[:, None])
    any_j = overlap.any(axis=1)
    jlo = jnp.where(any_j, jnp.argmax(overlap, axis=1).astype(jnp.int32), 1)
    jhi = jnp.where(
        any_j,
        (NBn - 1 - jnp.argmax(overlap[:, ::-1], axis=1)).astype(jnp.int32), 0)

    E_pad = -(-E // (NWORK * ECH)) * (NWORK * ECH)
    padE = E_pad - E
    src1 = jnp.pad(edge_index1[0], (0, padE))
    dst1 = jnp.pad(edge_index1[1], (0, padE), constant_values=N)
    src2 = jnp.pad(edge_index2[0], (0, padE))
    dst2 = jnp.pad(edge_index2[1], (0, padE), constant_values=N)

    r1 = lambda a: a.reshape(1, -1)
    emb_b2 = r1(emb_b)
    bg2, bv2 = r1(bg), r1(bv)

    h1p, A1, B1 = _embed_call(xp1, emb_W, emb_b2, layers[0]['Wm1'],
                              r1(layers[0]['bm1']), N, NP, H)
    h2p, A2, B2 = _embed_call(xp2, emb_W, emb_b2, layers[0]['Wm1'],
                              r1(layers[0]['bm1']), N, NP, H)

    c1o, c2o = _build_sc_deg(NP, E_pad)(dst1, dst2)
    deg1r = (c1o[0, :, 0] + c1o[1, :, 0]).reshape(NBn, 1, BN)
    deg2r = (c2o[0, :, 0] + c2o[1, :, 0]).reshape(NBn, 1, BN)

    num_layers = len(layers)
    sc_msg = _build_sc_messages(NP, E_pad, H)
    for l, p in enumerate(layers):
        out1, out2 = sc_msg(A1, B1, A2, B2, src1, dst1, src2, dst2)

        cnt1, cnt2, cs1, cs2 = _counts_call(b1r, b2r, h1p, h2p, NP, H)
        M1, M2 = _passA_call(jlo, jhi, h1p, h2p, b1r, b2r, NP, H)
        M1r = M1.reshape(NBn, 1, BN)
        M2r = M2.reshape(NBn, 1, BN)
        O1, O2, S1, S2 = _passB_call(jlo, jhi, h1p, h2p, b1r, b2r, M1r, M2r,
                                     NP, H)
        S1r = S1.reshape(NBn, 1, BN)
        S2r = S2.reshape(NBn, 1, BN)

        last = (l == num_layers - 1)
        Wm1n = None if last else layers[l + 1]['Wm1']
        bm1n = None if last else r1(layers[l + 1]['bm1'])
        res1 = _update_call(h1p, out1, O1, S1r, deg1r, b1r, cnt2, cs2,
                            p['Wm2'], r1(p['bm2']), p['Wu1'], r1(p['bu1']),
                            p['Wu2'], r1(p['bu2']), Wm1n, bm1n, N, NP, H)
        res2 = _update_call(h2p, out2, O2, S2r, deg2r, b2r, cnt1, cs1,
                            p['Wm2'], r1(p['bm2']), p['Wu1'], r1(p['bu1']),
                            p['Wu2'], r1(p['bu2']), Wm1n, bm1n, N, NP, H)
        if last:
            h1p, = res1
            h2p, = res2
        else:
            h1p, A1, B1 = res1
            h2p, A2, B2 = res2

    R1, R2 = _readout_call(h1p, h2p, b1r, b2r, Wg, bg2, Wv, bv2, NP, H)
    return (R1, R2)
